# baseline (speedup 1.0000x reference)
import functools
import math

import jax
import jax.numpy as jnp
from jax import lax
from jax.experimental import pallas as pl
from jax.experimental.pallas import tpu as pltpu
from jax.experimental.pallas import tpu_sc as plsc

N = 32768
D_ATTR = 16
HID = 64
B = 16
EPS = 1e-6
LN_EPS = math.log(1e-6)
LN2 = math.log(2.0)

BLK = 16384
NB = N // BLK

NS = 16
LANES = 16
CHUNK = N // NS
NVEC = CHUNK // LANES


def _node_grad_body(post_ref, attrt_ref, w1_ref, b1_ref, w2_ref, gs_ref):
    w1 = w1_ref[...].astype(jnp.bfloat16)
    w1p = w1[0:3, :]
    w1a = w1[3:, :]
    b1c = jnp.transpose(b1_ref[...])
    pre_t = (lax.dot_general(w1p, post_ref[...].astype(jnp.bfloat16),
                             (((0,), (0,)), ((), ())),
                             preferred_element_type=jnp.float32)
             + lax.dot_general(w1a, attrt_ref[...].astype(jnp.bfloat16),
                               (((0,), (0,)), ((), ())),
                               preferred_element_type=jnp.float32)
             + b1c)
    t = jnp.tanh(pre_t)
    u_t = ((1.0 - t * t) * w2_ref[...]).astype(jnp.bfloat16)
    g_t = lax.dot_general(w1p, u_t, (((1,), (0,)), ((), ())),
                          preferred_element_type=jnp.float32)
    gs_ref[...] = jnp.sum(g_t * g_t, axis=0, keepdims=True)[None]


def _node_grads_squared(pos_t, attr_t, w1, b1r, w2):
    return pl.pallas_call(
        _node_grad_body,
        grid=(NB,),
        in_specs=[
            pl.BlockSpec((3, BLK), lambda i: (0, i)),
            pl.BlockSpec((D_ATTR, BLK), lambda i: (0, i)),
            pl.BlockSpec((3 + D_ATTR, HID), lambda i: (0, 0)),
            pl.BlockSpec((1, HID), lambda i: (0, 0)),
            pl.BlockSpec((HID, 1), lambda i: (0, 0)),
        ],
        out_specs=pl.BlockSpec((1, 1, BLK), lambda i: (i, 0, 0)),
        out_shape=jax.ShapeDtypeStruct((NB, 1, BLK), jnp.float32),
        compiler_params=pltpu.CompilerParams(
            dimension_semantics=("parallel",)),
    )(pos_t, attr_t, w1, b1r, w2)


def _log_f32(x):
    bits = lax.bitcast_convert_type(x, jnp.int32)
    e = lax.shift_right_logical(bits, 23) - 127
    mbits = (bits & 0x007FFFFF) | 0x3F800000
    m = lax.bitcast_convert_type(mbits, jnp.float32)
    big = m > 1.4142135
    m = jnp.where(big, m * 0.5, m)
    ef = (e + jnp.where(big, 1, 0)).astype(jnp.float32)
    s = (m - 1.0) / (m + 1.0)
    s2 = s * s
    poly = 1.0 + s2 * (1.0 / 3.0 + s2 * (1.0 / 5.0 + s2 * (1.0 / 7.0 + s2 * (1.0 / 9.0))))
    return ef * LN2 + 2.0 * s * poly


NSUB = 4
TBL = LANES * B


def _segment_bias_body(gs_hbm, bat_hbm, out_hbm, gs_v, bat_v, table_v, shared,
                       sem1, sem2):
    sid = lax.axis_index("s")
    cp1 = pltpu.async_copy(gs_hbm.at[pl.ds(sid * CHUNK, CHUNK)], gs_v, sem1)
    cp2 = pltpu.async_copy(bat_hbm.at[pl.ds(sid * CHUNK, CHUNK)], bat_v, sem2)

    def zstep(j, carry):
        table_v[pl.ds(j * LANES, LANES)] = jnp.zeros((LANES,), jnp.float32)
        return carry

    lax.fori_loop(0, NSUB * B, zstep, 0)
    lane_base = lax.iota(jnp.int32, LANES) * B
    cp1.wait()
    cp2.wait()

    def step(j, carry):
        for k in range(NSUB):
            jj = j * NSUB + k
            v = gs_v[pl.ds(jj * LANES, LANES)]
            seg = bat_v[pl.ds(jj * LANES, LANES)]
            plsc.addupdate_scatter(table_v, [k * TBL + lane_base + seg], v)
        return carry

    lax.fori_loop(0, NVEC // NSUB, step, 0)

    def rstep(j, acc):
        return acc + table_v[pl.ds(j * LANES, LANES)]

    acc = lax.fori_loop(1, NSUB * LANES, rstep, table_v[pl.ds(0, LANES)])
    table_v[pl.ds(0, LANES)] = acc
    pltpu.sync_copy(table_v.at[pl.ds(0, LANES)],
                    shared.at[pl.ds(sid * LANES, LANES)])

    plsc.subcore_barrier()

    @pl.when(sid == 0)
    def _finalize():
        pltpu.sync_copy(shared, table_v.at[pl.ds(0, NS * LANES)])
        total = table_v[pl.ds(0, LANES)]
        for r in range(1, NS):
            total = total + table_v[pl.ds(r * LANES, LANES)]
        bias = LN_EPS - _log_f32(total + EPS)
        table_v[pl.ds(0, B)] = bias
        pltpu.sync_copy(table_v.at[pl.ds(0, B)], out_hbm)


@functools.cache
def _segment_bias():
    return pl.kernel(
        _segment_bias_body,
        out_type=jax.ShapeDtypeStruct((B,), jnp.float32),
        mesh=plsc.VectorSubcoreMesh(core_axis_name="c", subcore_axis_name="s",
                                    num_cores=1, num_subcores=NS),
        compiler_params=pltpu.CompilerParams(needs_layout_passes=False),
        scratch_types=[
            pltpu.VMEM((CHUNK,), jnp.float32),
            pltpu.VMEM((CHUNK,), jnp.int32),
            pltpu.VMEM((NSUB * TBL,), jnp.float32),
            pltpu.VMEM_SHARED((NS * LANES,), jnp.float32),
            pltpu.SemaphoreType.DMA,
            pltpu.SemaphoreType.DMA,
        ],
    )


def kernel(positions, node_attrs, batch, W1, b1, W2):
    gs = _node_grads_squared(positions.T, node_attrs.T, W1,
                             b1.reshape(1, HID), W2)
    return _segment_bias()(gs.reshape(N), batch)


# speedup vs baseline: 1.0060x; 1.0060x over previous
import functools
import math

import jax
import jax.numpy as jnp
from jax import lax
from jax.experimental import pallas as pl
from jax.experimental.pallas import tpu as pltpu
from jax.experimental.pallas import tpu_sc as plsc

N = 32768
D_ATTR = 16
HID = 64
B = 16
EPS = 1e-6
LN_EPS = math.log(1e-6)
LN2 = math.log(2.0)

BLK = 16384
NB = N // BLK

NS = 16
LANES = 16
CHUNK = N // NS
NVEC = CHUNK // LANES


def _node_grad_body(post_ref, attrt_ref, w1_ref, b1_ref, w2_ref, gs_ref):
    w1 = w1_ref[...]
    w1p = w1[0:3, :]
    w1a = w1[3:, :]
    b1c = jnp.transpose(b1_ref[...])
    pre_t = (lax.dot_general(w1p, post_ref[...], (((0,), (0,)), ((), ())),
                             preferred_element_type=jnp.float32)
             + lax.dot_general(w1a, attrt_ref[...],
                               (((0,), (0,)), ((), ())),
                               preferred_element_type=jnp.float32)
             + b1c)
    t = jnp.tanh(pre_t)
    u_t = (1.0 - t * t) * w2_ref[...]
    g_t = lax.dot_general(w1p, u_t, (((1,), (0,)), ((), ())),
                          preferred_element_type=jnp.float32)
    gs_ref[...] = jnp.sum(g_t * g_t, axis=0, keepdims=True)[None]


def _node_grads_squared(pos_t, attr_t, w1, b1r, w2):
    return pl.pallas_call(
        _node_grad_body,
        grid=(NB,),
        in_specs=[
            pl.BlockSpec((3, BLK), lambda i: (0, i)),
            pl.BlockSpec((D_ATTR, BLK), lambda i: (0, i)),
            pl.BlockSpec((3 + D_ATTR, HID), lambda i: (0, 0)),
            pl.BlockSpec((1, HID), lambda i: (0, 0)),
            pl.BlockSpec((HID, 1), lambda i: (0, 0)),
        ],
        out_specs=pl.BlockSpec((1, 1, BLK), lambda i: (i, 0, 0)),
        out_shape=jax.ShapeDtypeStruct((NB, 1, BLK), jnp.float32),
        compiler_params=pltpu.CompilerParams(
            dimension_semantics=("parallel",)),
    )(pos_t, attr_t, w1, b1r, w2)


def _log_f32(x):
    bits = lax.bitcast_convert_type(x, jnp.int32)
    e = lax.shift_right_logical(bits, 23) - 127
    mbits = (bits & 0x007FFFFF) | 0x3F800000
    m = lax.bitcast_convert_type(mbits, jnp.float32)
    big = m > 1.4142135
    m = jnp.where(big, m * 0.5, m)
    ef = (e + jnp.where(big, 1, 0)).astype(jnp.float32)
    s = (m - 1.0) / (m + 1.0)
    s2 = s * s
    poly = 1.0 + s2 * (1.0 / 3.0 + s2 * (1.0 / 5.0 + s2 * (1.0 / 7.0 + s2 * (1.0 / 9.0))))
    return ef * LN2 + 2.0 * s * poly


NSUB = 4
TBL = LANES * B


def _segment_bias_body(gs_hbm, bat_hbm, out_hbm, gs_v, bat_v, table_v, shared,
                       sem1, sem2):
    sid = lax.axis_index("s")
    cp1 = pltpu.async_copy(gs_hbm.at[pl.ds(sid * CHUNK, CHUNK)], gs_v, sem1)
    cp2 = pltpu.async_copy(bat_hbm.at[pl.ds(sid * CHUNK, CHUNK)], bat_v, sem2)

    def zstep(j, carry):
        table_v[pl.ds(j * LANES, LANES)] = jnp.zeros((LANES,), jnp.float32)
        return carry

    lax.fori_loop(0, NSUB * B, zstep, 0)
    lane_base = lax.iota(jnp.int32, LANES) * B
    cp1.wait()
    cp2.wait()

    def step(j, carry):
        for k in range(NSUB):
            jj = j * NSUB + k
            v = gs_v[pl.ds(jj * LANES, LANES)]
            seg = bat_v[pl.ds(jj * LANES, LANES)]
            plsc.addupdate_scatter(table_v, [k * TBL + lane_base + seg], v)
        return carry

    lax.fori_loop(0, NVEC // NSUB, step, 0)

    def rstep(j, acc):
        return acc + table_v[pl.ds(j * LANES, LANES)]

    acc = lax.fori_loop(1, NSUB * LANES, rstep, table_v[pl.ds(0, LANES)])
    table_v[pl.ds(0, LANES)] = acc
    pltpu.sync_copy(table_v.at[pl.ds(0, LANES)],
                    shared.at[pl.ds(sid * LANES, LANES)])

    plsc.subcore_barrier()

    @pl.when(sid == 0)
    def _finalize():
        pltpu.sync_copy(shared, table_v.at[pl.ds(0, NS * LANES)])
        total = table_v[pl.ds(0, LANES)]
        for r in range(1, NS):
            total = total + table_v[pl.ds(r * LANES, LANES)]
        bias = LN_EPS - _log_f32(total + EPS)
        table_v[pl.ds(0, B)] = bias
        pltpu.sync_copy(table_v.at[pl.ds(0, B)], out_hbm)


@functools.cache
def _segment_bias():
    return pl.kernel(
        _segment_bias_body,
        out_type=jax.ShapeDtypeStruct((B,), jnp.float32),
        mesh=plsc.VectorSubcoreMesh(core_axis_name="c", subcore_axis_name="s",
                                    num_cores=1, num_subcores=NS),
        compiler_params=pltpu.CompilerParams(needs_layout_passes=False),
        scratch_types=[
            pltpu.VMEM((CHUNK,), jnp.float32),
            pltpu.VMEM((CHUNK,), jnp.int32),
            pltpu.VMEM((NSUB * TBL,), jnp.float32),
            pltpu.VMEM_SHARED((NS * LANES,), jnp.float32),
            pltpu.SemaphoreType.DMA,
            pltpu.SemaphoreType.DMA,
        ],
    )


def kernel(positions, node_attrs, batch, W1, b1, W2):
    gs = _node_grads_squared(positions.T, node_attrs.T, W1,
                             b1.reshape(1, HID), W2)
    return _segment_bias()(gs.reshape(N), batch)
